# baseline (speedup 1.0000x reference)
import functools

import jax
import jax.numpy as jnp
from jax import lax
from jax.experimental import pallas as pl
from jax.experimental.pallas import tpu as pltpu
from jax.experimental.pallas import tpu_sc as plsc

_EMBED = 300
_SPLIT = 256
_TAIL = _EMBED - _SPLIT
_NC = 2
_NS = 16
_NW = _NC * _NS


@functools.partial(jax.jit, static_argnums=(3, 4))
def _gather(table, tail_tab, idx, n_per_w, chunk):
    n_total = idx.shape[0]
    ngroups = n_per_w // chunk // 2
    mesh = plsc.VectorSubcoreMesh(core_axis_name="c", subcore_axis_name="s")

    @functools.partial(
        pl.kernel,
        mesh=mesh,
        out_type=jax.ShapeDtypeStruct((n_total, _EMBED), jnp.float32),
        scratch_types=[
            pltpu.VMEM((n_per_w,), jnp.int32),
            pltpu.VMEM((chunk, _SPLIT), jnp.float32),
            pltpu.VMEM((chunk, _SPLIT), jnp.float32),
            pltpu.VMEM((chunk, 128), jnp.float32),
            pltpu.VMEM((chunk, 128), jnp.float32),
            pltpu.VMEM((chunk, _EMBED), jnp.float32),
            pltpu.VMEM((chunk, _EMBED), jnp.float32),
            pltpu.SemaphoreType.DMA,
            pltpu.SemaphoreType.DMA,
            pltpu.SemaphoreType.DMA,
            pltpu.SemaphoreType.DMA,
            pltpu.SemaphoreType.DMA,
            pltpu.SemaphoreType.DMA,
        ],
        compiler_params=pltpu.CompilerParams(needs_layout_passes=False),
    )
    def k(table_hbm, tail_hbm, idx_hbm, out_hbm,
          idx_v, a0, a1, t0, t1, w0, w1, ga0, ga1, gt0, gt1, ws0, ws1):
        wid = lax.axis_index("s") * _NC + lax.axis_index("c")
        base = wid * n_per_w
        pltpu.sync_copy(idx_hbm.at[pl.ds(base, n_per_w)], idx_v)
        head = table_hbm.at[:, pl.ds(0, _SPLIT)]
        abufs, tbufs, wbufs = (a0, a1), (t0, t1), (w0, w1)
        gas, gts, wss = (ga0, ga1), (gt0, gt1), (ws0, ws1)

        def gstart(c, b):
            isl = idx_v.at[pl.ds(c * chunk, chunk)]
            pltpu.async_copy(head.at[isl], abufs[b], gas[b])
            pltpu.async_copy(tail_hbm.at[isl], tbufs[b], gts[b])

        def gwait(b):
            pltpu.make_async_copy(head.at[pl.ds(0, chunk)], abufs[b], gas[b]).wait()
            pltpu.make_async_copy(tail_hbm.at[pl.ds(0, chunk)], tbufs[b], gts[b]).wait()

        def wstart(c, b):
            pltpu.async_copy(
                wbufs[b], out_hbm.at[pl.ds(base + c * chunk, chunk)], wss[b]
            )

        def wwait(b):
            pltpu.make_async_copy(
                wbufs[b], out_hbm.at[pl.ds(base, chunk)], wss[b]
            ).wait()

        def vcopy(b):
            a, t, w = abufs[b], tbufs[b], wbufs[b]

            def rcopy(i, carry):
                for p in range(_SPLIT // 16):
                    w[i, pl.ds(p * 16, 16)] = a[i, pl.ds(p * 16, 16)]
                w[i, pl.ds(_SPLIT, 16)] = t[i, pl.ds(0, 16)]
                w[i, pl.ds(_SPLIT + 16, 16)] = t[i, pl.ds(16, 16)]
                lanes = lax.iota(jnp.int32, 16)
                plsc.store_scatter(
                    w,
                    [jnp.full((16,), i, jnp.int32), _SPLIT + 32 + lanes],
                    t[i, pl.ds(32, 16)],
                    mask=lanes < _TAIL - 32,
                )
                return carry

            lax.fori_loop(0, chunk, rcopy, 0, unroll=2)

        gstart(0, 0)
        gstart(1, 1)

        def body(g, carry):
            for b in range(2):
                c = 2 * g + b
                gwait(b)

                @pl.when(g >= 1)
                def _():
                    wwait(b)

                vcopy(b)

                @pl.when(g < ngroups - 1)
                def _():
                    gstart(c + 2, b)

                wstart(c, b)
            return carry

        lax.fori_loop(0, ngroups, body, 0)
        wwait(0)
        wwait(1)

    return k(table, tail_tab, idx)


def kernel(sequence, segment_label, token_table):
    B, L = sequence.shape
    n_total = B * L
    seq = sequence.reshape(n_total).astype(jnp.int32)
    tail_tab = jnp.pad(token_table[:, _SPLIT:], ((0, 0), (0, 128 - _TAIL)))
    out = _gather(token_table, tail_tab, seq, n_total // _NW, 64)
    return out.reshape(B, L, _EMBED)
